# baseline (speedup 1.0000x reference)
import functools

import jax
import jax.numpy as jnp
from jax import lax
from jax.experimental import pallas as pl
from jax.experimental.pallas import tpu as pltpu
from jax.experimental.pallas import tpu_sc as plsc

B = 4096
HW = 128
D = 128
T = 24
K = 8
K2 = 16
FEAT = 512
BS = 128

NC, NS, L = 2, 16, 16
NW = NC * NS
BPW = B // NW


def _pool_feat(img_ref, bs):
    parts = [img_ref[:, 0, pl.Slice(k, 16, 8), :] for k in range(8)]
    acc = parts[0]
    for pk in parts[1:]:
        acc = acc + pk
    x = acc.reshape(bs * 16, HW) * (1.0 / 64.0)
    r = jax.lax.broadcasted_iota(jnp.int32, (HW, 16), 0) // 8
    c = jax.lax.broadcasted_iota(jnp.int32, (HW, 16), 1)
    pc = (r == c).astype(jnp.float32)
    p = jnp.dot(x, pc, preferred_element_type=jnp.float32,
                precision=jax.lax.Precision.HIGHEST)
    p3 = p.reshape(bs, 16, 16)
    return jnp.concatenate([p3[:, i, :] for i in range(16)], axis=1)


def _pool_feat_tree(img_ref, bs):
    x = img_ref[...].reshape(bs * 16, 8, HW).sum(axis=1) * (1.0 / 64.0)
    r = jax.lax.broadcasted_iota(jnp.int32, (HW, 16), 0) // 8
    c = jax.lax.broadcasted_iota(jnp.int32, (HW, 16), 1)
    pc = (r == c).astype(jnp.float32)
    p = jnp.dot(x, pc, preferred_element_type=jnp.float32,
                precision=jax.lax.Precision.HIGHEST)
    p3 = p.reshape(bs, 16, 16)
    return jnp.concatenate([p3[:, i, :] for i in range(16)], axis=1)


def _tc_body(l_ref, r_ref, we_ref, be_ref, wl_ref, bl_ref, p_ref,
             logits_o, emb_o, sims_o, *, bs):
    fl = _pool_feat(l_ref, bs)
    fr = _pool_feat_tree(r_ref, bs)
    feat = jnp.concatenate([fl, fr], axis=1)

    emb = jnp.dot(feat, we_ref[...], preferred_element_type=jnp.float32) + be_ref[...]
    emb_o[...] = emb
    logits_o[...] = jnp.dot(emb, wl_ref[...], preferred_element_type=jnp.float32) + bl_ref[...]

    n1 = jnp.sqrt(jnp.sum(emb * emb, axis=1, keepdims=True))
    emb_n = emb / jnp.maximum(n1, 1e-12)
    na = jnp.maximum(jnp.sqrt(jnp.sum(emb_n * emb_n, axis=1, keepdims=True)), 1e-8)
    emb_s = emb_n / na

    p0 = p_ref[...]
    pn1 = jnp.sqrt(jnp.sum(p0 * p0, axis=1, keepdims=True))
    pn = p0 / jnp.maximum(pn1, 1e-12)
    nb = jnp.maximum(jnp.sqrt(jnp.sum(pn * pn, axis=1, keepdims=True)), 1e-8)
    ps = pn / nb

    s = jax.lax.dot_general(
        emb_s, ps, (((1,), (1,)), ((), ())),
        preferred_element_type=jnp.float32,
        precision=jax.lax.Precision.HIGHEST)
    padm = (jax.lax.broadcasted_iota(jnp.int32, s.shape, 1) % K2) >= K
    sims_o[...] = jnp.where(padm, -3e38, s)


def _sc_body(sims_hbm, chr_hbm, dists_hbm, mind_hbm, midx_hbm,
             chr_v, idx_v, rows_v, dist_v, mind_v, midx_v, sem):
    wid = lax.axis_index("s") * NC + lax.axis_index("c")
    base = wid * BPW
    pltpu.sync_copy(chr_hbm.at[pl.ds(base, BPW)], chr_v)
    io16 = lax.iota(jnp.int32, L)
    for j in range(BPW // L):
        c16 = chr_v[pl.ds(j * L, L)]
        idx_v[pl.ds(j * L, L)] = (base + j * L + io16) * T + c16
    pltpu.async_copy(sims_hbm.at[idx_v], rows_v, sem).wait()
    gdn = lax.GatherDimensionNumbers(offset_dims=(), collapsed_slice_dims=(0,),
                                     start_index_map=(0,))

    def _perm(x, stride):
        return lax.gather(x, (io16 ^ stride)[:, None], gdn, slice_sizes=(1,),
                          mode=lax.GatherScatterMode.PROMISE_IN_BOUNDS)

    def _allmin(x):
        for stp in (8, 4, 2, 1):
            x = jnp.minimum(x, _perm(x, stp))
        return x

    for j0 in range(BPW // L):
        m16 = jnp.zeros((L,), jnp.float32)
        mi16 = jnp.zeros((L,), jnp.int32)
        for t in range(L):
            dv = 1.0 - rows_v[j0 * L + t]
            dist_v[j0 * L + t] = dv
            ms = _allmin(dv)
            mis = _allmin(jnp.where(dv == ms, io16, L))
            sel = io16 == t
            m16 = jnp.where(sel, ms, m16)
            mi16 = jnp.where(sel, mis, mi16)
        mind_v[pl.ds(j0 * L, L)] = m16
        midx_v[pl.ds(j0 * L, L)] = mi16
    pltpu.sync_copy(dist_v, dists_hbm.at[pl.ds(base, BPW)])
    pltpu.sync_copy(mind_v, mind_hbm.at[pl.ds(base, BPW)])
    pltpu.sync_copy(midx_v, midx_hbm.at[pl.ds(base, BPW)])


@jax.jit
def kernel(left_image, right_image, chr_idx, W_embed, b_embed, W_logits, b_logits, prototypes):
    bs = BS
    grid = (B // bs,)
    ppad = jnp.concatenate(
        [prototypes, jnp.zeros((T, K2 - K, D), jnp.float32)], axis=1)
    p2 = ppad.reshape(T * K2, D)

    logits, emb, sims = pl.pallas_call(
        functools.partial(_tc_body, bs=bs),
        grid=grid,
        in_specs=[
            pl.BlockSpec((bs, 1, HW, HW), lambda i: (i, 0, 0, 0)),
            pl.BlockSpec((bs, 1, HW, HW), lambda i: (i, 0, 0, 0)),
            pl.BlockSpec((FEAT, D), lambda i: (0, 0)),
            pl.BlockSpec((1, D), lambda i: (0, 0)),
            pl.BlockSpec((D, T), lambda i: (0, 0)),
            pl.BlockSpec((1, T), lambda i: (0, 0)),
            pl.BlockSpec((T * K2, D), lambda i: (0, 0)),
        ],
        out_specs=[
            pl.BlockSpec((bs, T), lambda i: (i, 0)),
            pl.BlockSpec((bs, D), lambda i: (i, 0)),
            pl.BlockSpec((bs, T * K2), lambda i: (i, 0)),
        ],
        out_shape=[
            jax.ShapeDtypeStruct((B, T), jnp.float32),
            jax.ShapeDtypeStruct((B, D), jnp.float32),
            jax.ShapeDtypeStruct((B, T * K2), jnp.float32),
        ],
    )(left_image, right_image, W_embed, b_embed.reshape(1, D),
      W_logits, b_logits.reshape(1, T), p2)

    sims_tab = sims.reshape(B * T, K2)
    chr32 = chr_idx.astype(jnp.int32)

    sc = functools.partial(
        pl.kernel,
        mesh=plsc.VectorSubcoreMesh(core_axis_name="c", subcore_axis_name="s"),
        compiler_params=pltpu.CompilerParams(use_tc_tiling_on_sc=False),
        out_type=[
            jax.ShapeDtypeStruct((B, K2), jnp.float32),
            jax.ShapeDtypeStruct((B,), jnp.float32),
            jax.ShapeDtypeStruct((B,), jnp.int32),
        ],
        scratch_types=[
            pltpu.VMEM((BPW,), jnp.int32),
            pltpu.VMEM((BPW,), jnp.int32),
            pltpu.VMEM((BPW, K2), jnp.float32),
            pltpu.VMEM((BPW, K2), jnp.float32),
            pltpu.VMEM((BPW,), jnp.float32),
            pltpu.VMEM((BPW,), jnp.int32),
            pltpu.SemaphoreType.DMA,
        ],
    )(_sc_body)
    dists16, mind, midx = sc(sims_tab, chr32)

    return (logits, emb, dists16[:, :K], mind, midx, prototypes)


# speedup vs baseline: 1.0921x; 1.0921x over previous
import functools

import jax
import jax.numpy as jnp
from jax import lax
from jax.experimental import pallas as pl
from jax.experimental.pallas import tpu as pltpu
from jax.experimental.pallas import tpu_sc as plsc

B = 4096
HW = 128
D = 128
T = 24
K = 8
K2 = 16
FEAT = 512
BS = 128

NC, NS, L = 2, 16, 16
NW = NC * NS
BPW = B // NW


def _pool_feat(img_ref, bs):
    parts = [img_ref[:, 0, pl.Slice(k, 16, 8), :] for k in range(8)]
    acc = parts[0]
    for pk in parts[1:]:
        acc = acc + pk
    x = acc.reshape(bs * 16, HW) * (1.0 / 64.0)
    r = jax.lax.broadcasted_iota(jnp.int32, (HW, 16), 0) // 8
    c = jax.lax.broadcasted_iota(jnp.int32, (HW, 16), 1)
    pc = (r == c).astype(jnp.float32)
    p = jnp.dot(x, pc, preferred_element_type=jnp.float32,
                precision=jax.lax.Precision.HIGHEST)
    p3 = p.reshape(bs, 16, 16)
    return jnp.concatenate([p3[:, i, :] for i in range(16)], axis=1)


def _tc_body(l_ref, r_ref, we_ref, be_ref, wl_ref, bl_ref, p_ref,
             logits_o, emb_o, sims_o, *, bs):
    fl = _pool_feat(l_ref, bs)
    fr = _pool_feat(r_ref, bs)
    feat = jnp.concatenate([fl, fr], axis=1)

    emb = jnp.dot(feat, we_ref[...], preferred_element_type=jnp.float32) + be_ref[...]
    emb_o[...] = emb
    logits_o[...] = jnp.dot(emb, wl_ref[...], preferred_element_type=jnp.float32) + bl_ref[...]

    n1 = jnp.sqrt(jnp.sum(emb * emb, axis=1, keepdims=True))
    emb_n = emb / jnp.maximum(n1, 1e-12)
    na = jnp.maximum(jnp.sqrt(jnp.sum(emb_n * emb_n, axis=1, keepdims=True)), 1e-8)
    emb_s = emb_n / na

    p0 = p_ref[...]
    pn1 = jnp.sqrt(jnp.sum(p0 * p0, axis=1, keepdims=True))
    pn = p0 / jnp.maximum(pn1, 1e-12)
    nb = jnp.maximum(jnp.sqrt(jnp.sum(pn * pn, axis=1, keepdims=True)), 1e-8)
    ps = pn / nb

    s = jax.lax.dot_general(
        emb_s, ps, (((1,), (1,)), ((), ())),
        preferred_element_type=jnp.float32,
        precision=jax.lax.Precision.HIGHEST)
    padm = (jax.lax.broadcasted_iota(jnp.int32, s.shape, 1) % K2) >= K
    sims_o[...] = jnp.where(padm, -3e38, s)


def _sc_body(sims_hbm, chr_hbm, dists_hbm, mind_hbm, midx_hbm,
             chr_v, idx_v, rows_v, dist_v, mind_v, midx_v, sem):
    wid = lax.axis_index("s") * NC + lax.axis_index("c")
    base = wid * BPW
    pltpu.sync_copy(chr_hbm.at[pl.ds(base, BPW)], chr_v)
    io16 = lax.iota(jnp.int32, L)
    for j in range(BPW // L):
        c16 = chr_v[pl.ds(j * L, L)]
        idx_v[pl.ds(j * L, L)] = (base + j * L + io16) * T + c16
    pltpu.async_copy(sims_hbm.at[idx_v], rows_v, sem).wait()
    gdn = lax.GatherDimensionNumbers(offset_dims=(), collapsed_slice_dims=(0,),
                                     start_index_map=(0,))

    def _perm(x, stride):
        return lax.gather(x, (io16 ^ stride)[:, None], gdn, slice_sizes=(1,),
                          mode=lax.GatherScatterMode.PROMISE_IN_BOUNDS)

    def _allmin(x):
        for stp in (8, 4, 2, 1):
            x = jnp.minimum(x, _perm(x, stp))
        return x

    for j0 in range(BPW // L):
        m16 = jnp.zeros((L,), jnp.float32)
        mi16 = jnp.zeros((L,), jnp.int32)
        for t in range(L):
            dv = 1.0 - rows_v[j0 * L + t]
            dist_v[j0 * L + t] = dv
            ms = _allmin(dv)
            mis = _allmin(jnp.where(dv == ms, io16, L))
            sel = io16 == t
            m16 = jnp.where(sel, ms, m16)
            mi16 = jnp.where(sel, mis, mi16)
        mind_v[pl.ds(j0 * L, L)] = m16
        midx_v[pl.ds(j0 * L, L)] = mi16
    pltpu.sync_copy(dist_v, dists_hbm.at[pl.ds(base, BPW)])
    pltpu.sync_copy(mind_v, mind_hbm.at[pl.ds(base, BPW)])
    pltpu.sync_copy(midx_v, midx_hbm.at[pl.ds(base, BPW)])


@jax.jit
def kernel(left_image, right_image, chr_idx, W_embed, b_embed, W_logits, b_logits, prototypes):
    bs = BS
    grid = (B // bs,)
    ppad = jnp.concatenate(
        [prototypes, jnp.zeros((T, K2 - K, D), jnp.float32)], axis=1)
    p2 = ppad.reshape(T * K2, D)

    logits, emb, sims = pl.pallas_call(
        functools.partial(_tc_body, bs=bs),
        grid=grid,
        in_specs=[
            pl.BlockSpec((bs, 1, HW, HW), lambda i: (i, 0, 0, 0)),
            pl.BlockSpec((bs, 1, HW, HW), lambda i: (i, 0, 0, 0)),
            pl.BlockSpec((FEAT, D), lambda i: (0, 0)),
            pl.BlockSpec((1, D), lambda i: (0, 0)),
            pl.BlockSpec((D, T), lambda i: (0, 0)),
            pl.BlockSpec((1, T), lambda i: (0, 0)),
            pl.BlockSpec((T * K2, D), lambda i: (0, 0)),
        ],
        out_specs=[
            pl.BlockSpec((bs, T), lambda i: (i, 0)),
            pl.BlockSpec((bs, D), lambda i: (i, 0)),
            pl.BlockSpec((bs, T * K2), lambda i: (i, 0)),
        ],
        out_shape=[
            jax.ShapeDtypeStruct((B, T), jnp.float32),
            jax.ShapeDtypeStruct((B, D), jnp.float32),
            jax.ShapeDtypeStruct((B, T * K2), jnp.float32),
        ],
    )(left_image, right_image, W_embed, b_embed.reshape(1, D),
      W_logits, b_logits.reshape(1, T), p2)

    sims_tab = sims.reshape(B * T, K2)
    chr32 = chr_idx.astype(jnp.int32)

    sc = functools.partial(
        pl.kernel,
        mesh=plsc.VectorSubcoreMesh(core_axis_name="c", subcore_axis_name="s"),
        compiler_params=pltpu.CompilerParams(use_tc_tiling_on_sc=False),
        out_type=[
            jax.ShapeDtypeStruct((B, K2), jnp.float32),
            jax.ShapeDtypeStruct((B,), jnp.float32),
            jax.ShapeDtypeStruct((B,), jnp.int32),
        ],
        scratch_types=[
            pltpu.VMEM((BPW,), jnp.int32),
            pltpu.VMEM((BPW,), jnp.int32),
            pltpu.VMEM((BPW, K2), jnp.float32),
            pltpu.VMEM((BPW, K2), jnp.float32),
            pltpu.VMEM((BPW,), jnp.float32),
            pltpu.VMEM((BPW,), jnp.int32),
            pltpu.SemaphoreType.DMA,
        ],
    )(_sc_body)
    dists16, mind, midx = sc(sims_tab, chr32)

    return (logits, emb, dists16[:, :K], mind, midx, prototypes)


# speedup vs baseline: 1.1170x; 1.0228x over previous
import functools

import jax
import jax.numpy as jnp
from jax import lax
from jax.experimental import pallas as pl
from jax.experimental.pallas import tpu as pltpu
from jax.experimental.pallas import tpu_sc as plsc

B = 4096
HW = 128
D = 128
T = 24
K = 8
K2 = 16
FEAT = 512
BS = 128

NC, NS, L = 2, 16, 16
NW = NC * NS
BPW = B // NW


def _pool_feat(img_ref, bs, s1, s2):
    s1[...] = (img_ref[:, 0, pl.Slice(0, 64, 2), :]
               + img_ref[:, 0, pl.Slice(1, 64, 2), :]).reshape(bs * 64, HW)
    s2[...] = s1[pl.Slice(0, bs * 32, 2), :] + s1[pl.Slice(1, bs * 32, 2), :]
    x = (s2[pl.Slice(0, bs * 16, 2), :]
         + s2[pl.Slice(1, bs * 16, 2), :]) * (1.0 / 64.0)
    r = jax.lax.broadcasted_iota(jnp.int32, (HW, 16), 0) // 8
    c = jax.lax.broadcasted_iota(jnp.int32, (HW, 16), 1)
    pc = (r == c).astype(jnp.float32)
    p = jnp.dot(x, pc, preferred_element_type=jnp.float32,
                precision=jax.lax.Precision.HIGHEST)
    p3 = p.reshape(bs, 16, 16)
    return jnp.concatenate([p3[:, i, :] for i in range(16)], axis=1)


def _tc_body(l_ref, r_ref, we_ref, be_ref, wl_ref, bl_ref, p_ref,
             logits_o, emb_o, sims_o, s1, s2, *, bs):
    fl = _pool_feat(l_ref, bs, s1, s2)
    fr = _pool_feat(r_ref, bs, s1, s2)
    feat = jnp.concatenate([fl, fr], axis=1)

    emb = jnp.dot(feat, we_ref[...], preferred_element_type=jnp.float32) + be_ref[...]
    emb_o[...] = emb
    logits_o[...] = jnp.dot(emb, wl_ref[...], preferred_element_type=jnp.float32) + bl_ref[...]

    n1 = jnp.sqrt(jnp.sum(emb * emb, axis=1, keepdims=True))
    emb_n = emb / jnp.maximum(n1, 1e-12)
    na = jnp.maximum(jnp.sqrt(jnp.sum(emb_n * emb_n, axis=1, keepdims=True)), 1e-8)
    emb_s = emb_n / na

    p0 = p_ref[...]
    pn1 = jnp.sqrt(jnp.sum(p0 * p0, axis=1, keepdims=True))
    pn = p0 / jnp.maximum(pn1, 1e-12)
    nb = jnp.maximum(jnp.sqrt(jnp.sum(pn * pn, axis=1, keepdims=True)), 1e-8)
    ps = pn / nb

    s = jax.lax.dot_general(
        emb_s, ps, (((1,), (1,)), ((), ())),
        preferred_element_type=jnp.float32,
        precision=jax.lax.Precision.HIGHEST)
    padm = (jax.lax.broadcasted_iota(jnp.int32, s.shape, 1) % K2) >= K
    sims_o[...] = jnp.where(padm, -3e38, s)


def _sc_body(sims_hbm, chr_hbm, dists_hbm, mind_hbm, midx_hbm,
             chr_v, idx_v, rows_v, dist_v, mind_v, midx_v, sem):
    wid = lax.axis_index("s") * NC + lax.axis_index("c")
    base = wid * BPW
    pltpu.sync_copy(chr_hbm.at[pl.ds(base, BPW)], chr_v)
    io16 = lax.iota(jnp.int32, L)
    for j in range(BPW // L):
        c16 = chr_v[pl.ds(j * L, L)]
        idx_v[pl.ds(j * L, L)] = (base + j * L + io16) * T + c16
    pltpu.async_copy(sims_hbm.at[idx_v], rows_v, sem).wait()
    gdn = lax.GatherDimensionNumbers(offset_dims=(), collapsed_slice_dims=(0,),
                                     start_index_map=(0,))

    def _perm(x, stride):
        return lax.gather(x, (io16 ^ stride)[:, None], gdn, slice_sizes=(1,),
                          mode=lax.GatherScatterMode.PROMISE_IN_BOUNDS)

    def _allmin(x):
        for stp in (8, 4, 2, 1):
            x = jnp.minimum(x, _perm(x, stp))
        return x

    for j0 in range(BPW // L):
        m16 = jnp.zeros((L,), jnp.float32)
        mi16 = jnp.zeros((L,), jnp.int32)
        for t in range(L):
            dv = 1.0 - rows_v[j0 * L + t]
            dist_v[j0 * L + t] = dv
            ms = _allmin(dv)
            mis = _allmin(jnp.where(dv == ms, io16, L))
            sel = io16 == t
            m16 = jnp.where(sel, ms, m16)
            mi16 = jnp.where(sel, mis, mi16)
        mind_v[pl.ds(j0 * L, L)] = m16
        midx_v[pl.ds(j0 * L, L)] = mi16
    pltpu.sync_copy(dist_v, dists_hbm.at[pl.ds(base, BPW)])
    pltpu.sync_copy(mind_v, mind_hbm.at[pl.ds(base, BPW)])
    pltpu.sync_copy(midx_v, midx_hbm.at[pl.ds(base, BPW)])


@jax.jit
def kernel(left_image, right_image, chr_idx, W_embed, b_embed, W_logits, b_logits, prototypes):
    bs = BS
    grid = (B // bs,)
    ppad = jnp.concatenate(
        [prototypes, jnp.zeros((T, K2 - K, D), jnp.float32)], axis=1)
    p2 = ppad.reshape(T * K2, D)

    logits, emb, sims = pl.pallas_call(
        functools.partial(_tc_body, bs=bs),
        grid=grid,
        in_specs=[
            pl.BlockSpec((bs, 1, HW, HW), lambda i: (i, 0, 0, 0)),
            pl.BlockSpec((bs, 1, HW, HW), lambda i: (i, 0, 0, 0)),
            pl.BlockSpec((FEAT, D), lambda i: (0, 0)),
            pl.BlockSpec((1, D), lambda i: (0, 0)),
            pl.BlockSpec((D, T), lambda i: (0, 0)),
            pl.BlockSpec((1, T), lambda i: (0, 0)),
            pl.BlockSpec((T * K2, D), lambda i: (0, 0)),
        ],
        out_specs=[
            pl.BlockSpec((bs, T), lambda i: (i, 0)),
            pl.BlockSpec((bs, D), lambda i: (i, 0)),
            pl.BlockSpec((bs, T * K2), lambda i: (i, 0)),
        ],
        out_shape=[
            jax.ShapeDtypeStruct((B, T), jnp.float32),
            jax.ShapeDtypeStruct((B, D), jnp.float32),
            jax.ShapeDtypeStruct((B, T * K2), jnp.float32),
        ],
        scratch_shapes=[
            pltpu.VMEM((BS * 64, HW), jnp.float32),
            pltpu.VMEM((BS * 32, HW), jnp.float32),
        ],
    )(left_image, right_image, W_embed, b_embed.reshape(1, D),
      W_logits, b_logits.reshape(1, T), p2)

    sims_tab = sims.reshape(B * T, K2)
    chr32 = chr_idx.astype(jnp.int32)

    sc = functools.partial(
        pl.kernel,
        mesh=plsc.VectorSubcoreMesh(core_axis_name="c", subcore_axis_name="s"),
        compiler_params=pltpu.CompilerParams(use_tc_tiling_on_sc=False),
        out_type=[
            jax.ShapeDtypeStruct((B, K2), jnp.float32),
            jax.ShapeDtypeStruct((B,), jnp.float32),
            jax.ShapeDtypeStruct((B,), jnp.int32),
        ],
        scratch_types=[
            pltpu.VMEM((BPW,), jnp.int32),
            pltpu.VMEM((BPW,), jnp.int32),
            pltpu.VMEM((BPW, K2), jnp.float32),
            pltpu.VMEM((BPW, K2), jnp.float32),
            pltpu.VMEM((BPW,), jnp.float32),
            pltpu.VMEM((BPW,), jnp.int32),
            pltpu.SemaphoreType.DMA,
        ],
    )(_sc_body)
    dists16, mind, midx = sc(sims_tab, chr32)

    return (logits, emb, dists16[:, :K], mind, midx, prototypes)
